# baseline (device time: 8102 ns/iter reference)
import jax
import jax.numpy as jnp
from jax import lax
from jax.experimental import pallas as pl
from jax.experimental.pallas import tpu as pltpu

N_DEV = 8

_FAR_FIRST = (6, 2, 5, 7, 3, 1, 4)


def kernel(x):
    m_per, n = x.shape

    def body(x_hbm, out_ref, xv_ref, comm_ref, copy_sem, send_sems, recv_sems):
        my = lax.axis_index("i")

        cp = pltpu.make_async_copy(x_hbm, xv_ref, copy_sem)
        cp.start()

        barrier_sem = pltpu.get_barrier_semaphore()
        for k in _FAR_FIRST:
            tgt = lax.rem(my + k, N_DEV)
            pl.semaphore_signal(
                barrier_sem, inc=1,
                device_id=(tgt,), device_id_type=pl.DeviceIdType.MESH,
            )

        cp.wait()
        comm_ref[pl.ds(my, 1), :] = jnp.sum(xv_ref[:], axis=0, keepdims=True)

        pl.semaphore_wait(barrier_sem, N_DEV - 1)

        sends = []
        for k in _FAR_FIRST:
            tgt = lax.rem(my + k, N_DEV)
            rdma = pltpu.make_async_remote_copy(
                src_ref=comm_ref.at[pl.ds(my, 1)],
                dst_ref=comm_ref.at[pl.ds(my, 1)],
                send_sem=send_sems.at[k - 1],
                recv_sem=recv_sems.at[k - 1],
                device_id=(tgt,),
                device_id_type=pl.DeviceIdType.MESH,
            )
            rdma.start()
            sends.append(rdma)

        for k in range(1, N_DEV):
            src = lax.rem(my - k + N_DEV, N_DEV)
            recv = pltpu.make_async_remote_copy(
                src_ref=comm_ref.at[pl.ds(src, 1)],
                dst_ref=comm_ref.at[pl.ds(src, 1)],
                send_sem=send_sems.at[k - 1],
                recv_sem=recv_sems.at[k - 1],
                device_id=(src,),
                device_id_type=pl.DeviceIdType.MESH,
            )
            recv.wait_recv()

        out_ref[:, :] = jnp.sum(comm_ref[:, :], axis=0, keepdims=True)

        for rdma in sends:
            rdma.wait_send()

    return pl.pallas_call(
        body,
        out_shape=jax.ShapeDtypeStruct((1, n), jnp.float32),
        in_specs=[pl.BlockSpec(memory_space=pl.ANY)],
        out_specs=pl.BlockSpec(memory_space=pltpu.VMEM),
        scratch_shapes=[
            pltpu.VMEM((m_per, n), jnp.float32),
            pltpu.VMEM((N_DEV, n), jnp.float32),
            pltpu.SemaphoreType.DMA,
            pltpu.SemaphoreType.DMA((N_DEV - 1,)),
            pltpu.SemaphoreType.DMA((N_DEV - 1,)),
        ],
        compiler_params=pltpu.CompilerParams(collective_id=0),
    )(x)


# device time: 8076 ns/iter; 1.0032x vs baseline; 1.0032x over previous
import jax
import jax.numpy as jnp
from jax import lax
from jax.experimental import pallas as pl
from jax.experimental.pallas import tpu as pltpu

N_DEV = 8

_FAR_FIRST = (6, 2, 5, 7, 3, 1, 4)


def kernel(x):
    m_per, n = x.shape

    def body(x_hbm, out_ref, xv_ref, part_ref, comm_ref,
             copy_sem, send_sems, recv_sems):
        my = lax.axis_index("i")

        cp = pltpu.make_async_copy(x_hbm, xv_ref, copy_sem)
        cp.start()

        barrier_sem = pltpu.get_barrier_semaphore()
        for k in _FAR_FIRST:
            tgt = lax.rem(my + k, N_DEV)
            pl.semaphore_signal(
                barrier_sem, inc=1,
                device_id=(tgt,), device_id_type=pl.DeviceIdType.MESH,
            )

        cp.wait()
        part_ref[:, :] = jnp.sum(xv_ref[:], axis=0, keepdims=True)

        pl.semaphore_wait(barrier_sem, N_DEV - 1)

        sends = []
        for k in _FAR_FIRST:
            tgt = lax.rem(my + k, N_DEV)
            rdma = pltpu.make_async_remote_copy(
                src_ref=part_ref,
                dst_ref=comm_ref.at[N_DEV - 1 - k],
                send_sem=send_sems.at[N_DEV - 1 - k],
                recv_sem=recv_sems.at[N_DEV - 1 - k],
                device_id=(tgt,),
                device_id_type=pl.DeviceIdType.MESH,
            )
            rdma.start()
            sends.append(rdma)

        for j in range(N_DEV - 1):
            src = lax.rem(my - (j + 1) + N_DEV, N_DEV)
            recv = pltpu.make_async_remote_copy(
                src_ref=part_ref,
                dst_ref=comm_ref.at[j],
                send_sem=send_sems.at[j],
                recv_sem=recv_sems.at[j],
                device_id=(src,),
                device_id_type=pl.DeviceIdType.MESH,
            )
            recv.wait_recv()

        out_ref[:, :] = part_ref[:, :] + jnp.sum(
            comm_ref[:, :, :], axis=0
        )

        for rdma in sends:
            rdma.wait_send()

    return pl.pallas_call(
        body,
        out_shape=jax.ShapeDtypeStruct((1, n), jnp.float32),
        in_specs=[pl.BlockSpec(memory_space=pl.ANY)],
        out_specs=pl.BlockSpec(memory_space=pltpu.VMEM),
        scratch_shapes=[
            pltpu.VMEM((m_per, n), jnp.float32),
            pltpu.VMEM((1, n), jnp.float32),
            pltpu.VMEM((N_DEV - 1, 1, n), jnp.float32),
            pltpu.SemaphoreType.DMA,
            pltpu.SemaphoreType.DMA((N_DEV - 1,)),
            pltpu.SemaphoreType.DMA((N_DEV - 1,)),
        ],
        compiler_params=pltpu.CompilerParams(collective_id=0),
    )(x)
